# baseline (speedup 1.0000x reference)
import functools

import numpy as np
import jax
import jax.numpy as jnp
from jax import lax
from jax.experimental import pallas as pl
from jax.experimental.pallas import tpu as pltpu
from jax.experimental.pallas import tpu_sc as plsc

_S = 512
_B = 16
_BROWS = 24
_UNROLL = 4


def _rinv() -> np.ndarray:
    col = np.arange(_S, dtype=np.float32)[None, :]
    row = np.arange(_S, dtype=np.float32)[:, None]
    px = col - 256.0
    py = row - 256.0
    r = np.sqrt(px * px + py * py).astype(np.float32)
    r[256, 256] = 1.0
    return (np.float32(1.0) / r).reshape(_S, 32, 16)


_RINV_NP = _rinv()

_mesh = plsc.VectorSubcoreMesh(core_axis_name="c", subcore_axis_name="s")

_GDN = lax.GatherDimensionNumbers(
    offset_dims=(), collapsed_slice_dims=(0,), start_index_map=(0,))


@functools.partial(
    pl.kernel,
    mesh=_mesh,
    out_type=(
        jax.ShapeDtypeStruct((_B, _S * _S), jnp.float32),
        jax.ShapeDtypeStruct((256, 16), jnp.float32),
    ),
    scratch_types=[
        pltpu.VMEM((_S, 2, 16), jnp.float32),
        pltpu.VMEM((_S, 2, 16), jnp.float32),
        pltpu.VMEM((_S, 2, 16), jnp.float32),
        pltpu.VMEM((_BROWS * _S,), jnp.float32),
        pltpu.VMEM((8, 16), jnp.float32),
    ],
    compiler_params=pltpu.CompilerParams(
        needs_layout_passes=False, use_tc_tiling_on_sc=False),
)
def _scatter_kernel(x4, xt4, r4, out_hbm, maxes_hbm, xs, xts, rs, band, maxbuf):
    wid = lax.axis_index("s") * 2 + lax.axis_index("c")
    g0 = jnp.minimum(wid, 30)

    rb = jnp.where(wid == 0, 0, 16 * wid + 8)
    band_base = rb * _S
    nrows = jnp.where(wid == 0, 24, jnp.where(wid == 31, 8, 16))
    nwords = nrows * _S

    lane = lax.iota(jnp.int32, 16)
    lane_f = lane.astype(jnp.float32)
    lane15 = lane == 15
    perm = jnp.minimum(lane + 1, 15)[:, None]
    colbase_f = (16 * g0).astype(jnp.float32)
    pxv = [(colbase_f + (16.0 * v)) + lane_f - 256.0 for v in range(2)]
    zeros16 = jnp.zeros((16,), jnp.float32)

    def _zero(i, c):
        band[pl.ds(i * 16, 16)] = zeros16
        return c

    lax.fori_loop(0, _BROWS * _S // 16, _zero, 0)

    pltpu.sync_copy(r4.at[:, pl.ds(g0, 2), :], rs)

    mvec = zeros16

    for b in range(_B):
        pltpu.sync_copy(x4.at[b, :, pl.ds(g0, 2), :], xs)
        pltpu.sync_copy(xt4.at[b, :, pl.ds(g0, 2), :], xts)

        def _rows(i, c):
            r0 = i * _UNROLL
            base_f = r0.astype(jnp.float32) - 256.0
            for j in range(_UNROLL):
                rr = r0 + j
                pyv = jnp.full((16,), base_f + jnp.float32(j))
                for v in range(2):
                    xv = xs[rr, v, :]
                    iv = rs[rr, v, :]
                    vals = xts[rr, v, :]
                    e = jnp.exp(-xv)
                    s = 1.0 / (1.0 + e)
                    k = 0.4 * s + 0.8
                    tx = (k * pxv[v]) * iv
                    orow = ((pxv[v] - tx) + 256.0 + 0.5).astype(jnp.int32)
                    ty = (k * pyv) * iv
                    ocol = ((pyv - ty) + 256.0 + 0.5).astype(jnp.int32)
                    local = orow * _S + ocol - band_base
                    key = local * 16 + lane
                    ks, vs = plsc.sort_key_val(key, vals)
                    locs = lax.shift_right_arithmetic(ks, 4)
                    nxt = lax.gather(
                        locs, perm, _GDN, (1,),
                        mode=lax.GatherScatterMode.PROMISE_IN_BOUNDS)
                    ok = (((locs != nxt) | lane15)
                          & (locs >= 0) & (locs < nwords))
                    plsc.store_scatter(band, [locs], vs, mask=ok)
            return c

        lax.fori_loop(0, _S // _UNROLL, _rows, 0)

        @pl.when(wid == 0)
        def _():
            pltpu.sync_copy(band.at[pl.ds(0, 24 * _S)],
                            out_hbm.at[b, pl.ds(0, 24 * _S)])

        @pl.when((wid > 0) & (wid < 31))
        def _():
            pltpu.sync_copy(band.at[pl.ds(0, 16 * _S)],
                            out_hbm.at[b, pl.ds(band_base, 16 * _S)])

        @pl.when(wid == 31)
        def _():
            pltpu.sync_copy(band.at[pl.ds(0, 8 * _S)],
                            out_hbm.at[b, pl.ds(504 * _S, 8 * _S)])

        def _drain(i, m):
            seg = band[pl.ds(i * 16, 16)]
            band[pl.ds(i * 16, 16)] = zeros16
            return jnp.maximum(m, seg)

        mvec = lax.fori_loop(0, nwords // 16, _drain, mvec)

    for i in range(8):
        maxbuf[i, :] = mvec
    pltpu.sync_copy(maxbuf, maxes_hbm.at[pl.ds(wid * 8, 8), :])


def _norm_body(o_ref, mx_ref, out_ref):
    m = jnp.max(mx_ref[...])
    out_ref[...] = o_ref[...] / (m + 1e-9)


def _normalize(out3, maxes):
    return pl.pallas_call(
        _norm_body,
        grid=(_B,),
        in_specs=[
            pl.BlockSpec((1, _S, _S), lambda i: (i, 0, 0)),
            pl.BlockSpec((256, 16), lambda i: (0, 0)),
        ],
        out_specs=pl.BlockSpec((1, _S, _S), lambda i: (i, 0, 0)),
        out_shape=jax.ShapeDtypeStruct((_B, _S, _S), jnp.float32),
    )(out3, maxes)


def kernel(x, x_true):
    x4 = x.reshape(_B, _S, 32, 16)
    xt4 = x_true.reshape(_B, _S, 32, 16)
    r4 = jnp.asarray(_RINV_NP)
    out_flat, maxes = _scatter_kernel(x4, xt4, r4)
    out = _normalize(out_flat.reshape(_B, _S, _S), maxes)
    return out.reshape(_B, 1, _S, _S)


# speedup vs baseline: 1.9108x; 1.9108x over previous
import functools

import numpy as np
import jax
import jax.numpy as jnp
from jax import lax
from jax.experimental import pallas as pl
from jax.experimental.pallas import tpu as pltpu
from jax.experimental.pallas import tpu_sc as plsc

_S = 512
_B = 16
_BROWS = 24
_UNROLL = 4


def _rinv() -> np.ndarray:
    col = np.arange(_S, dtype=np.float32)[None, :]
    row = np.arange(_S, dtype=np.float32)[:, None]
    px = col - 256.0
    py = row - 256.0
    r = np.sqrt(px * px + py * py).astype(np.float32)
    r[256, 256] = 1.0
    return (np.float32(1.0) / r).reshape(_S, 32, 16)


_RINV_NP = _rinv()

_mesh = plsc.VectorSubcoreMesh(core_axis_name="c", subcore_axis_name="s")

_GDN = lax.GatherDimensionNumbers(
    offset_dims=(), collapsed_slice_dims=(0,), start_index_map=(0,))


@functools.partial(
    pl.kernel,
    mesh=_mesh,
    out_type=(
        jax.ShapeDtypeStruct((_B, _S * _S), jnp.float32),
        jax.ShapeDtypeStruct((256, 16), jnp.float32),
    ),
    scratch_types=[
        pltpu.VMEM((_S, 2, 16), jnp.float32),
        pltpu.VMEM((_S, 2, 16), jnp.float32),
        pltpu.VMEM((_S, 2, 16), jnp.float32),
        pltpu.VMEM((_BROWS * _S,), jnp.float32),
        pltpu.VMEM((8, 16), jnp.float32),
    ],
    compiler_params=pltpu.CompilerParams(
        needs_layout_passes=False, use_tc_tiling_on_sc=False),
)
def _scatter_kernel(x4, xt4, r4, out_hbm, maxes_hbm, xs, xts, rs, band, maxbuf):
    wid = lax.axis_index("s") * 2 + lax.axis_index("c")
    g0 = jnp.minimum(wid, 30)

    rb = jnp.where(wid == 0, 0, 16 * wid + 8)
    band_base = rb * _S
    nrows = jnp.where(wid == 0, 24, jnp.where(wid == 31, 8, 16))
    nwords = nrows * _S

    lane = lax.iota(jnp.int32, 16)
    lane_f = lane.astype(jnp.float32)
    lane15 = lane == 15
    perm = jnp.minimum(lane + 1, 15)[:, None]
    colbase_f = (16 * g0).astype(jnp.float32)
    pxv = [(colbase_f + (16.0 * v)) + lane_f - 256.0 for v in range(2)]
    zeros16 = jnp.zeros((16,), jnp.float32)

    def _zero(i, c):
        band[pl.ds(i * 16, 16)] = zeros16
        return c

    lax.fori_loop(0, _BROWS * _S // 16, _zero, 0)

    pltpu.sync_copy(r4.at[:, pl.ds(g0, 2), :], rs)

    mvec = zeros16

    for b in range(_B):
        pltpu.sync_copy(x4.at[b, :, pl.ds(g0, 2), :], xs)
        pltpu.sync_copy(xt4.at[b, :, pl.ds(g0, 2), :], xts)

        def _rows(i, c):
            r0 = i * _UNROLL
            base_f = r0.astype(jnp.float32) - 256.0
            pending = []
            for j in range(_UNROLL):
                rr = r0 + j
                pyv = jnp.full((16,), base_f + jnp.float32(j))
                for v in range(2):
                    xv = xs[rr, v, :]
                    iv = rs[rr, v, :]
                    vals = xts[rr, v, :]
                    e = jnp.exp(-xv)
                    s = 1.0 / (1.0 + e)
                    k = 0.4 * s + 0.8
                    tx = (k * pxv[v]) * iv
                    orow = ((pxv[v] - tx) + 256.0 + 0.5).astype(jnp.int32)
                    ty = (k * pyv) * iv
                    ocol = ((pyv - ty) + 256.0 + 0.5).astype(jnp.int32)
                    local = orow * _S + ocol - band_base
                    key = local * 16 + lane
                    ks, vs = plsc.sort_key_val(key, vals)
                    locs = lax.shift_right_arithmetic(ks, 4)
                    nxt = lax.gather(
                        locs, perm, _GDN, (1,),
                        mode=lax.GatherScatterMode.PROMISE_IN_BOUNDS)
                    ok = (((locs != nxt) | lane15)
                          & (locs >= 0) & (locs < nwords))
                    pending.append((locs, vs, ok))
            for locs, vs, ok in pending:
                plsc.store_scatter(band, [locs], vs, mask=ok)
            return c

        lax.fori_loop(0, _S // _UNROLL, _rows, 0)

        @pl.when(wid == 0)
        def _():
            pltpu.sync_copy(band.at[pl.ds(0, 24 * _S)],
                            out_hbm.at[b, pl.ds(0, 24 * _S)])

        @pl.when((wid > 0) & (wid < 31))
        def _():
            pltpu.sync_copy(band.at[pl.ds(0, 16 * _S)],
                            out_hbm.at[b, pl.ds(band_base, 16 * _S)])

        @pl.when(wid == 31)
        def _():
            pltpu.sync_copy(band.at[pl.ds(0, 8 * _S)],
                            out_hbm.at[b, pl.ds(504 * _S, 8 * _S)])

        def _drain(i, m):
            segs = [band[pl.ds(i * 64 + 16 * u, 16)] for u in range(4)]
            for u in range(4):
                band[pl.ds(i * 64 + 16 * u, 16)] = zeros16
            for seg in segs:
                m = jnp.maximum(m, seg)
            return m

        mvec = lax.fori_loop(0, nwords // 64, _drain, mvec)

    for i in range(8):
        maxbuf[i, :] = mvec
    pltpu.sync_copy(maxbuf, maxes_hbm.at[pl.ds(wid * 8, 8), :])


def _norm_body(o_ref, mx_ref, out_ref):
    m = jnp.max(mx_ref[...])
    out_ref[...] = o_ref[...] / (m + 1e-9)


def _normalize(out3, maxes):
    return pl.pallas_call(
        _norm_body,
        grid=(_B,),
        in_specs=[
            pl.BlockSpec((1, _S, _S), lambda i: (i, 0, 0)),
            pl.BlockSpec((256, 16), lambda i: (0, 0)),
        ],
        out_specs=pl.BlockSpec((1, _S, _S), lambda i: (i, 0, 0)),
        out_shape=jax.ShapeDtypeStruct((_B, _S, _S), jnp.float32),
    )(out3, maxes)


def kernel(x, x_true):
    x4 = x.reshape(_B, _S, 32, 16)
    xt4 = x_true.reshape(_B, _S, 32, 16)
    r4 = jnp.asarray(_RINV_NP)
    out_flat, maxes = _scatter_kernel(x4, xt4, r4)
    out = _normalize(out_flat.reshape(_B, _S, _S), maxes)
    return out.reshape(_B, 1, _S, _S)


# speedup vs baseline: 2.2289x; 1.1665x over previous
import functools

import numpy as np
import jax
import jax.numpy as jnp
from jax import lax
from jax.experimental import pallas as pl
from jax.experimental.pallas import tpu as pltpu
from jax.experimental.pallas import tpu_sc as plsc

_S = 512
_B = 16
_BROWS = 24
_UNROLL = 4


def _rinv() -> np.ndarray:
    col = np.arange(_S, dtype=np.float32)[None, :]
    row = np.arange(_S, dtype=np.float32)[:, None]
    px = col - 256.0
    py = row - 256.0
    r = np.sqrt(px * px + py * py).astype(np.float32)
    r[256, 256] = 1.0
    return (np.float32(1.0) / r).reshape(_S, 32, 16)


_RINV_NP = _rinv()

_mesh = plsc.VectorSubcoreMesh(core_axis_name="c", subcore_axis_name="s")

_GDN = lax.GatherDimensionNumbers(
    offset_dims=(), collapsed_slice_dims=(0,), start_index_map=(0,))


@functools.partial(
    pl.kernel,
    mesh=_mesh,
    out_type=(
        jax.ShapeDtypeStruct((_B, _S * _S), jnp.float32),
        jax.ShapeDtypeStruct((256, 16), jnp.float32),
    ),
    scratch_types=[
        pltpu.VMEM((2, _S, 2, 16), jnp.float32),
        pltpu.VMEM((2, _S, 2, 16), jnp.float32),
        pltpu.VMEM((_S, 2, 16), jnp.float32),
        pltpu.VMEM((_BROWS * _S,), jnp.float32),
        pltpu.VMEM((8, 16), jnp.float32),
        pltpu.SemaphoreType.DMA,
        pltpu.SemaphoreType.DMA,
    ],
    compiler_params=pltpu.CompilerParams(
        needs_layout_passes=False, use_tc_tiling_on_sc=False),
)
def _scatter_kernel(x4, xt4, r4, out_hbm, maxes_hbm, xs2, xts2, rs, band,
                    maxbuf, sem0, sem1):
    wid = lax.axis_index("s") * 2 + lax.axis_index("c")
    g0 = jnp.minimum(wid, 30)

    rb = jnp.where(wid == 0, 0, 16 * wid + 8)
    band_base = rb * _S
    nrows = jnp.where(wid == 0, 24, jnp.where(wid == 31, 8, 16))
    nwords = nrows * _S

    lane = lax.iota(jnp.int32, 16)
    lane_f = lane.astype(jnp.float32)
    lane15 = lane == 15
    perm = jnp.minimum(lane + 1, 15)[:, None]
    colbase_f = (16 * g0).astype(jnp.float32)
    pxv = [(colbase_f + (16.0 * v)) + lane_f - 256.0 for v in range(2)]
    zeros16 = jnp.zeros((16,), jnp.float32)
    _BIAS = 1 << 18
    lane_bias = lane + (_BIAS * 16 - band_base * 16)
    nwords_u = nwords.astype(jnp.uint32)

    def _zero(i, c):
        band[pl.ds(i * 16, 16)] = zeros16
        return c

    lax.fori_loop(0, _BROWS * _S // 16, _zero, 0)

    pltpu.sync_copy(r4.at[:, pl.ds(g0, 2), :], rs)

    mvec = zeros16

    sems = [sem0, sem1]
    cps = [None, None]
    cps[0] = (
        pltpu.async_copy(x4.at[0, :, pl.ds(g0, 2), :], xs2.at[0], sem0),
        pltpu.async_copy(xt4.at[0, :, pl.ds(g0, 2), :], xts2.at[0], sem0),
    )

    for b in range(_B):
        cur = b % 2
        for cp in cps[cur]:
            cp.wait()
        if b + 1 < _B:
            nxt_buf = (b + 1) % 2
            cps[nxt_buf] = (
                pltpu.async_copy(x4.at[b + 1, :, pl.ds(g0, 2), :],
                                 xs2.at[nxt_buf], sems[nxt_buf]),
                pltpu.async_copy(xt4.at[b + 1, :, pl.ds(g0, 2), :],
                                 xts2.at[nxt_buf], sems[nxt_buf]),
            )
        xs = xs2.at[cur]
        xts = xts2.at[cur]

        def _rows(i, c):
            r0 = i * _UNROLL
            base_f = r0.astype(jnp.float32) - 256.0
            pending = []
            for j in range(_UNROLL):
                rr = r0 + j
                pyv = jnp.full((16,), base_f + jnp.float32(j))
                for v in range(2):
                    xv = xs[rr, v, :]
                    iv = rs[rr, v, :]
                    vals = xts[rr, v, :]
                    e = jnp.exp(-xv)
                    s = 1.0 / (1.0 + e)
                    k = 0.4 * s + 0.8
                    tx = (k * pxv[v]) * iv
                    orow = ((pxv[v] - tx) + 256.0 + 0.5).astype(jnp.int32)
                    ty = (k * pyv) * iv
                    ocol = ((pyv - ty) + 256.0 + 0.5).astype(jnp.int32)
                    tgt = orow * _S + ocol
                    key = plsc.bitcast(tgt * 16 + lane_bias, jnp.uint32)
                    ks, vs = plsc.sort_key_val(key, vals)
                    locs_b = lax.shift_right_arithmetic(
                        plsc.bitcast(ks, jnp.int32), 4)
                    nxt = lax.gather(
                        locs_b, perm, _GDN, (1,),
                        mode=lax.GatherScatterMode.PROMISE_IN_BOUNDS)
                    localu = plsc.bitcast(locs_b - _BIAS, jnp.uint32)
                    ok = ((locs_b != nxt) | lane15) & (localu < nwords_u)
                    pending.append((plsc.bitcast(localu, jnp.int32), vs, ok))
            for locs, vs, ok in pending:
                plsc.store_scatter(band, [locs], vs, mask=ok)
            return c

        lax.fori_loop(0, _S // _UNROLL, _rows, 0)

        @pl.when(wid == 0)
        def _():
            pltpu.sync_copy(band.at[pl.ds(0, 24 * _S)],
                            out_hbm.at[b, pl.ds(0, 24 * _S)])

        @pl.when((wid > 0) & (wid < 31))
        def _():
            pltpu.sync_copy(band.at[pl.ds(0, 16 * _S)],
                            out_hbm.at[b, pl.ds(band_base, 16 * _S)])

        @pl.when(wid == 31)
        def _():
            pltpu.sync_copy(band.at[pl.ds(0, 8 * _S)],
                            out_hbm.at[b, pl.ds(504 * _S, 8 * _S)])

        def _drain(i, m):
            segs = [band[pl.ds(i * 64 + 16 * u, 16)] for u in range(4)]
            for u in range(4):
                band[pl.ds(i * 64 + 16 * u, 16)] = zeros16
            for seg in segs:
                m = jnp.maximum(m, seg)
            return m

        mvec = lax.fori_loop(0, nwords // 64, _drain, mvec)

    for i in range(8):
        maxbuf[i, :] = mvec
    pltpu.sync_copy(maxbuf, maxes_hbm.at[pl.ds(wid * 8, 8), :])


def _norm_body(o_ref, mx_ref, out_ref):
    m = jnp.max(mx_ref[...])
    out_ref[...] = o_ref[...] / (m + 1e-9)


def _normalize(out3, maxes):
    return pl.pallas_call(
        _norm_body,
        grid=(_B,),
        in_specs=[
            pl.BlockSpec((1, _S, _S), lambda i: (i, 0, 0)),
            pl.BlockSpec((256, 16), lambda i: (0, 0)),
        ],
        out_specs=pl.BlockSpec((1, _S, _S), lambda i: (i, 0, 0)),
        out_shape=jax.ShapeDtypeStruct((_B, _S, _S), jnp.float32),
    )(out3, maxes)


def kernel(x, x_true):
    x4 = x.reshape(_B, _S, 32, 16)
    xt4 = x_true.reshape(_B, _S, 32, 16)
    r4 = jnp.asarray(_RINV_NP)
    out_flat, maxes = _scatter_kernel(x4, xt4, r4)
    out = _normalize(out_flat.reshape(_B, _S, _S), maxes)
    return out.reshape(_B, 1, _S, _S)


# speedup vs baseline: 3.4030x; 1.5268x over previous
import functools

import numpy as np
import jax
import jax.numpy as jnp
from jax import lax
from jax.experimental import pallas as pl
from jax.experimental.pallas import tpu as pltpu
from jax.experimental.pallas import tpu_sc as plsc

_S = 512
_B = 16
_BROWS = 24
_UNROLL = 4


def _rinv() -> np.ndarray:
    col = np.arange(_S, dtype=np.float32)[None, :]
    row = np.arange(_S, dtype=np.float32)[:, None]
    px = col - 256.0
    py = row - 256.0
    r = np.sqrt(px * px + py * py).astype(np.float32)
    r[256, 256] = 1.0
    return np.float32(1.0) / r


_RINV_NP = _rinv()

_mesh = plsc.VectorSubcoreMesh(core_axis_name="c", subcore_axis_name="s")

_GDN = lax.GatherDimensionNumbers(
    offset_dims=(), collapsed_slice_dims=(0,), start_index_map=(0,))


@functools.partial(
    pl.kernel,
    mesh=_mesh,
    out_type=(
        jax.ShapeDtypeStruct((_B, _S * _S), jnp.float32),
        jax.ShapeDtypeStruct((256, 16), jnp.float32),
    ),
    scratch_types=[
        pltpu.VMEM((2, _S, 32), jnp.float32),
        pltpu.VMEM((2, _S, 32), jnp.float32),
        pltpu.VMEM((_S, 32), jnp.float32),
        pltpu.VMEM((_BROWS * _S,), jnp.float32),
        pltpu.VMEM((8, 16), jnp.float32),
        pltpu.SemaphoreType.DMA,
        pltpu.SemaphoreType.DMA,
    ],
    compiler_params=pltpu.CompilerParams(
        needs_layout_passes=False, use_tc_tiling_on_sc=False),
)
def _scatter_kernel(x4, xt4, r4, out_hbm, maxes_hbm, xs2, xts2, rs, band,
                    maxbuf, sem0, sem1):
    wid = lax.axis_index("s") * 2 + lax.axis_index("c")
    g0 = jnp.minimum(wid, 30)

    rb = jnp.where(wid == 0, 0, 16 * wid + 8)
    band_base = rb * _S
    nrows = jnp.where(wid == 0, 24, jnp.where(wid == 31, 8, 16))
    nwords = nrows * _S

    lane = lax.iota(jnp.int32, 16)
    lane_f = lane.astype(jnp.float32)
    lane15 = lane == 15
    perm = jnp.minimum(lane + 1, 15)[:, None]
    colbase_f = (16 * g0).astype(jnp.float32)
    pxv = [(colbase_f + (16.0 * v)) + lane_f - 256.0 for v in range(2)]
    zeros16 = jnp.zeros((16,), jnp.float32)
    _BIAS = 1 << 18
    lane_bias = lane + (_BIAS * 16 - band_base * 16)
    nwords_u = nwords.astype(jnp.uint32)

    def _zero(i, c):
        band[pl.ds(i * 16, 16)] = zeros16
        return c

    lax.fori_loop(0, _BROWS * _S // 16, _zero, 0)

    pltpu.sync_copy(r4.at[:, pl.ds(16 * g0, 32)], rs)

    mvec = zeros16

    sems = [sem0, sem1]
    cps = [None, None]
    cps[0] = (
        pltpu.async_copy(x4.at[0, :, pl.ds(16 * g0, 32)], xs2.at[0], sem0),
        pltpu.async_copy(xt4.at[0, :, pl.ds(16 * g0, 32)], xts2.at[0], sem0),
    )

    for b in range(_B):
        cur = b % 2
        for cp in cps[cur]:
            cp.wait()
        if b + 1 < _B:
            nxt_buf = (b + 1) % 2
            cps[nxt_buf] = (
                pltpu.async_copy(x4.at[b + 1, :, pl.ds(16 * g0, 32)],
                                 xs2.at[nxt_buf], sems[nxt_buf]),
                pltpu.async_copy(xt4.at[b + 1, :, pl.ds(16 * g0, 32)],
                                 xts2.at[nxt_buf], sems[nxt_buf]),
            )
        xs = xs2.at[cur]
        xts = xts2.at[cur]

        def _rows(i, c):
            r0 = i * _UNROLL
            base_f = r0.astype(jnp.float32) - 256.0
            pending = []
            for j in range(_UNROLL):
                rr = r0 + j
                pyv = jnp.full((16,), base_f + jnp.float32(j))
                for v in range(2):
                    xv = xs[rr, pl.ds(16 * v, 16)]
                    iv = rs[rr, pl.ds(16 * v, 16)]
                    vals = xts[rr, pl.ds(16 * v, 16)]
                    e = jnp.exp(-xv)
                    s = 1.0 / (1.0 + e)
                    k = 0.4 * s + 0.8
                    tx = (k * pxv[v]) * iv
                    orow = ((pxv[v] - tx) + 256.0 + 0.5).astype(jnp.int32)
                    ty = (k * pyv) * iv
                    ocol = ((pyv - ty) + 256.0 + 0.5).astype(jnp.int32)
                    tgt = orow * _S + ocol
                    key = plsc.bitcast(tgt * 16 + lane_bias, jnp.uint32)
                    ks, vs = plsc.sort_key_val(key, vals)
                    locs_b = lax.shift_right_arithmetic(
                        plsc.bitcast(ks, jnp.int32), 4)
                    nxt = lax.gather(
                        locs_b, perm, _GDN, (1,),
                        mode=lax.GatherScatterMode.PROMISE_IN_BOUNDS)
                    localu = plsc.bitcast(locs_b - _BIAS, jnp.uint32)
                    ok = ((locs_b != nxt) | lane15) & (localu < nwords_u)
                    pending.append((plsc.bitcast(localu, jnp.int32), vs, ok))
            for locs, vs, ok in pending:
                plsc.store_scatter(band, [locs], vs, mask=ok)
            return c

        lax.fori_loop(0, _S // _UNROLL, _rows, 0)

        @pl.when(wid == 0)
        def _():
            pltpu.sync_copy(band.at[pl.ds(0, 24 * _S)],
                            out_hbm.at[b, pl.ds(0, 24 * _S)])

        @pl.when((wid > 0) & (wid < 31))
        def _():
            pltpu.sync_copy(band.at[pl.ds(0, 16 * _S)],
                            out_hbm.at[b, pl.ds(band_base, 16 * _S)])

        @pl.when(wid == 31)
        def _():
            pltpu.sync_copy(band.at[pl.ds(0, 8 * _S)],
                            out_hbm.at[b, pl.ds(504 * _S, 8 * _S)])

        def _drain(i, m):
            segs = [band[pl.ds(i * 64 + 16 * u, 16)] for u in range(4)]
            for u in range(4):
                band[pl.ds(i * 64 + 16 * u, 16)] = zeros16
            for seg in segs:
                m = jnp.maximum(m, seg)
            return m

        mvec = lax.fori_loop(0, nwords // 64, _drain, mvec)

    for i in range(8):
        maxbuf[i, :] = mvec
    pltpu.sync_copy(maxbuf, maxes_hbm.at[pl.ds(wid * 8, 8), :])


def _norm_body(o_ref, mx_ref, out_ref):
    m = jnp.max(mx_ref[...])
    out_ref[...] = o_ref[...] / (m + 1e-9)


def _normalize(out3, maxes):
    return pl.pallas_call(
        _norm_body,
        grid=(_B,),
        in_specs=[
            pl.BlockSpec((1, _S, _S), lambda i: (i, 0, 0)),
            pl.BlockSpec((256, 16), lambda i: (0, 0)),
        ],
        out_specs=pl.BlockSpec((1, _S, _S), lambda i: (i, 0, 0)),
        out_shape=jax.ShapeDtypeStruct((_B, _S, _S), jnp.float32),
    )(out3, maxes)


def kernel(x, x_true):
    x4 = x.reshape(_B, _S, _S)
    xt4 = x_true.reshape(_B, _S, _S)
    r4 = jnp.asarray(_RINV_NP)
    out_flat, maxes = _scatter_kernel(x4, xt4, r4)
    out = _normalize(out_flat.reshape(_B, _S, _S), maxes)
    return out.reshape(_B, 1, _S, _S)


# speedup vs baseline: 3.4066x; 1.0011x over previous
import functools

import numpy as np
import jax
import jax.numpy as jnp
from jax import lax
from jax.experimental import pallas as pl
from jax.experimental.pallas import tpu as pltpu
from jax.experimental.pallas import tpu_sc as plsc

_S = 512
_B = 16
_BROWS = 24
_UNROLL = 4


def _rinv() -> np.ndarray:
    col = np.arange(_S, dtype=np.float32)[None, :]
    row = np.arange(_S, dtype=np.float32)[:, None]
    px = col - 256.0
    py = row - 256.0
    r = np.sqrt(px * px + py * py).astype(np.float32)
    r[256, 256] = 1.0
    return np.float32(1.0) / r


_RINV_NP = _rinv()

_mesh = plsc.VectorSubcoreMesh(core_axis_name="c", subcore_axis_name="s")

_GDN = lax.GatherDimensionNumbers(
    offset_dims=(), collapsed_slice_dims=(0,), start_index_map=(0,))


@functools.partial(
    pl.kernel,
    mesh=_mesh,
    out_type=(
        jax.ShapeDtypeStruct((_B, _S * _S), jnp.float32),
        jax.ShapeDtypeStruct((256, 16), jnp.float32),
    ),
    scratch_types=[
        pltpu.VMEM((2, _S, 32), jnp.float32),
        pltpu.VMEM((2, _S, 32), jnp.float32),
        pltpu.VMEM((_S, 32), jnp.float32),
        pltpu.VMEM((_BROWS * _S,), jnp.float32),
        pltpu.VMEM((8, 16), jnp.float32),
        pltpu.SemaphoreType.DMA,
        pltpu.SemaphoreType.DMA,
    ],
    compiler_params=pltpu.CompilerParams(
        needs_layout_passes=False, use_tc_tiling_on_sc=False),
)
def _scatter_kernel(x4, xt4, r4, out_hbm, maxes_hbm, xs2, xts2, rs, band,
                    maxbuf, sem0, sem1):
    wid_raw = lax.axis_index("s") * 2 + lax.axis_index("c")
    wid = jnp.where(wid_raw == 30, 31, jnp.where(wid_raw == 31, 30, wid_raw))
    g0 = jnp.minimum(wid, 30)

    rb = jnp.where(wid == 0, 0, 16 * wid + 8)
    band_base = rb * _S
    nrows = jnp.where(wid == 0, 24, jnp.where(wid == 31, 8, 16))
    nwords = nrows * _S

    lane = lax.iota(jnp.int32, 16)
    lane_f = lane.astype(jnp.float32)
    lane15 = lane == 15
    perm = jnp.minimum(lane + 1, 15)[:, None]
    colbase_f = (16 * g0).astype(jnp.float32)
    pxv = [(colbase_f + (16.0 * v)) + lane_f - 256.0 for v in range(2)]
    zeros16 = jnp.zeros((16,), jnp.float32)
    _BIAS = 1 << 18
    lane_bias = lane + (_BIAS * 16 - band_base * 16)
    nwords_u = nwords.astype(jnp.uint32)

    def _zero(i, c):
        band[pl.ds(i * 16, 16)] = zeros16
        return c

    lax.fori_loop(0, _BROWS * _S // 16, _zero, 0)

    pltpu.sync_copy(r4.at[:, pl.ds(16 * g0, 32)], rs)

    mvec = zeros16

    sems = [sem0, sem1]
    cps = [None, None]
    cps[0] = (
        pltpu.async_copy(x4.at[0, :, pl.ds(16 * g0, 32)], xs2.at[0], sem0),
        pltpu.async_copy(xt4.at[0, :, pl.ds(16 * g0, 32)], xts2.at[0], sem0),
    )

    for b in range(_B):
        cur = b % 2
        for cp in cps[cur]:
            cp.wait()
        if b + 1 < _B:
            nxt_buf = (b + 1) % 2
            cps[nxt_buf] = (
                pltpu.async_copy(x4.at[b + 1, :, pl.ds(16 * g0, 32)],
                                 xs2.at[nxt_buf], sems[nxt_buf]),
                pltpu.async_copy(xt4.at[b + 1, :, pl.ds(16 * g0, 32)],
                                 xts2.at[nxt_buf], sems[nxt_buf]),
            )
        xs = xs2.at[cur]
        xts = xts2.at[cur]

        def _rows(i, c):
            r0 = i * _UNROLL
            base_f = r0.astype(jnp.float32) - 256.0
            pending = []
            for j in range(_UNROLL):
                rr = r0 + j
                pyv = jnp.full((16,), base_f + jnp.float32(j))
                for v in range(2):
                    xv = xs[rr, pl.ds(16 * v, 16)]
                    iv = rs[rr, pl.ds(16 * v, 16)]
                    vals = xts[rr, pl.ds(16 * v, 16)]
                    e = jnp.exp(-xv)
                    s = 1.0 / (1.0 + e)
                    k = 0.4 * s + 0.8
                    tx = (k * pxv[v]) * iv
                    orow = ((pxv[v] - tx) + 256.0 + 0.5).astype(jnp.int32)
                    ty = (k * pyv) * iv
                    ocol = ((pyv - ty) + 256.0 + 0.5).astype(jnp.int32)
                    tgt = orow * _S + ocol
                    key = plsc.bitcast(tgt * 16 + lane_bias, jnp.uint32)
                    ks, vs = plsc.sort_key_val(key, vals)
                    locs_b = lax.shift_right_arithmetic(
                        plsc.bitcast(ks, jnp.int32), 4)
                    nxt = lax.gather(
                        locs_b, perm, _GDN, (1,),
                        mode=lax.GatherScatterMode.PROMISE_IN_BOUNDS)
                    localu = plsc.bitcast(locs_b - _BIAS, jnp.uint32)
                    ok = ((locs_b != nxt) | lane15) & (localu < nwords_u)
                    pending.append((plsc.bitcast(localu, jnp.int32), vs, ok))
            for locs, vs, ok in pending:
                plsc.store_scatter(band, [locs], vs, mask=ok)
            return c

        lax.fori_loop(0, _S // _UNROLL, _rows, 0)

        @pl.when(wid == 0)
        def _():
            pltpu.sync_copy(band.at[pl.ds(0, 24 * _S)],
                            out_hbm.at[b, pl.ds(0, 24 * _S)])

        @pl.when((wid > 0) & (wid < 31))
        def _():
            pltpu.sync_copy(band.at[pl.ds(0, 16 * _S)],
                            out_hbm.at[b, pl.ds(band_base, 16 * _S)])

        @pl.when(wid == 31)
        def _():
            pltpu.sync_copy(band.at[pl.ds(0, 8 * _S)],
                            out_hbm.at[b, pl.ds(504 * _S, 8 * _S)])

        def _drain(i, m):
            segs = [band[pl.ds(i * 64 + 16 * u, 16)] for u in range(4)]
            for u in range(4):
                band[pl.ds(i * 64 + 16 * u, 16)] = zeros16
            for seg in segs:
                m = jnp.maximum(m, seg)
            return m

        mvec = lax.fori_loop(0, nwords // 64, _drain, mvec)

    for i in range(8):
        maxbuf[i, :] = mvec
    pltpu.sync_copy(maxbuf, maxes_hbm.at[pl.ds(wid * 8, 8), :])


def _norm_body(o_ref, mx_ref, out_ref):
    m = jnp.max(mx_ref[...])
    out_ref[...] = o_ref[...] / (m + 1e-9)


def _normalize(out3, maxes):
    return pl.pallas_call(
        _norm_body,
        grid=(_B,),
        in_specs=[
            pl.BlockSpec((1, _S, _S), lambda i: (i, 0, 0)),
            pl.BlockSpec((256, 16), lambda i: (0, 0)),
        ],
        out_specs=pl.BlockSpec((1, _S, _S), lambda i: (i, 0, 0)),
        out_shape=jax.ShapeDtypeStruct((_B, _S, _S), jnp.float32),
    )(out3, maxes)


def kernel(x, x_true):
    x4 = x.reshape(_B, _S, _S)
    xt4 = x_true.reshape(_B, _S, _S)
    r4 = jnp.asarray(_RINV_NP)
    out_flat, maxes = _scatter_kernel(x4, xt4, r4)
    out = _normalize(out_flat.reshape(_B, _S, _S), maxes)
    return out.reshape(_B, 1, _S, _S)
